# baseline (speedup 1.0000x reference)
import jax
import jax.numpy as jnp
from jax.experimental import pallas as pl
from jax.experimental.pallas import tpu as pltpu

N = 4096
H = 256
BM = 128
NB = N // BM

_DNT = (((1,), (1,)), ((), ()))


def _gcn_kernel(x_ref, Ap_ref, An_ref, Win_ref, bin_ref,
                Wp0_ref, bp0_ref, Wn0_ref, bn0_ref,
                Wp1_ref, bp1_ref, Wn1_ref, bn1_ref,
                out_ref, h0_ref, h1_ref):
    s = pl.program_id(0)

    @pl.when(s == 0)
    def _prep():
        h0_ref[...] = jnp.tanh(
            jax.lax.dot_general(x_ref[...], Win_ref[...], _DNT,
                                preferred_element_type=jnp.float32)
            + bin_ref[...]
        )

    def stripe(h, Wp, bp, Wn, bn):
        hp = jnp.dot(Ap_ref[...], h, preferred_element_type=jnp.float32)
        hn = jnp.dot(An_ref[...], h, preferred_element_type=jnp.float32)
        return jnp.tanh(
            jax.lax.dot_general(hp, Wp, _DNT,
                                preferred_element_type=jnp.float32)
            + bp
            + jax.lax.dot_general(hn, Wn, _DNT,
                                  preferred_element_type=jnp.float32)
            + bn
        )

    @pl.when(s < NB)
    def _layer0():
        t = stripe(h0_ref[...], Wp0_ref[...], bp0_ref[...],
                   Wn0_ref[...], bn0_ref[...])
        h1_ref[pl.ds(s * BM, BM), :] = t

    @pl.when(s >= NB)
    def _layer1():
        out_ref[...] = stripe(h1_ref[...], Wp1_ref[...], bp1_ref[...],
                              Wn1_ref[...], bn1_ref[...])


def _stripe_spec():
    return pl.BlockSpec((BM, N), lambda s: (s % NB, 0))


def _full_spec(shape):
    return pl.BlockSpec(shape, lambda s: (0,) * len(shape))


@jax.jit
def kernel(x, A_pos, A_neg, W_in, b_in, W_pos0, b_pos0, W_neg0, b_neg0,
           W_pos1, b_pos1, W_neg1, b_neg1):
    f32 = jnp.float32
    return pl.pallas_call(
        _gcn_kernel,
        grid=(2 * NB,),
        in_specs=[
            _full_spec((N, H)),
            _stripe_spec(),
            _stripe_spec(),
            _full_spec((H, H)),
            _full_spec((1, H)),
            _full_spec((H, H)),
            _full_spec((1, H)),
            _full_spec((H, H)),
            _full_spec((1, H)),
            _full_spec((H, H)),
            _full_spec((1, H)),
            _full_spec((H, H)),
            _full_spec((1, H)),
        ],
        out_specs=pl.BlockSpec((BM, H),
                               lambda s: (jnp.maximum(s - NB, 0), 0)),
        out_shape=jax.ShapeDtypeStruct((N, H), f32),
        scratch_shapes=[
            pltpu.VMEM((N, H), f32),
            pltpu.VMEM((N, H), f32),
        ],
    )(x, A_pos, A_neg, W_in, b_in.reshape(1, H),
      W_pos0, b_pos0.reshape(1, H), W_neg0, b_neg0.reshape(1, H),
      W_pos1, b_pos1.reshape(1, H), W_neg1, b_neg1.reshape(1, H))


# speedup vs baseline: 1.2115x; 1.2115x over previous
import jax
import jax.numpy as jnp
from jax.experimental import pallas as pl
from jax.experimental.pallas import tpu as pltpu

N = 4096
H = 256
BM = 256
NB = N // BM

_DNT = (((1,), (1,)), ((), ()))


def _gcn_kernel(x_ref, Ap_ref, An_ref, Win_ref, bin_ref,
                Wp0_ref, bp0_ref, Wn0_ref, bn0_ref,
                Wp1_ref, bp1_ref, Wn1_ref, bn1_ref,
                out_ref, h0_ref, h1_ref):
    s = pl.program_id(0)

    @pl.when(s == 0)
    def _prep():
        h0_ref[...] = jnp.tanh(
            jax.lax.dot_general(x_ref[...], Win_ref[...], _DNT,
                                preferred_element_type=jnp.float32)
            + bin_ref[...]
        )

    def stripe(h, Wp, bp, Wn, bn):
        hp = jnp.dot(Ap_ref[...], h, preferred_element_type=jnp.float32)
        hn = jnp.dot(An_ref[...], h, preferred_element_type=jnp.float32)
        return jnp.tanh(
            jax.lax.dot_general(hp, Wp, _DNT,
                                preferred_element_type=jnp.float32)
            + bp
            + jax.lax.dot_general(hn, Wn, _DNT,
                                  preferred_element_type=jnp.float32)
            + bn
        )

    @pl.when(s < NB)
    def _layer0():
        t = stripe(h0_ref[...], Wp0_ref[...], bp0_ref[...],
                   Wn0_ref[...], bn0_ref[...])
        h1_ref[pl.ds(s * BM, BM), :] = t

    @pl.when(s >= NB)
    def _layer1():
        out_ref[...] = stripe(h1_ref[...], Wp1_ref[...], bp1_ref[...],
                              Wn1_ref[...], bn1_ref[...])


def _stripe_spec():
    return pl.BlockSpec((BM, N), lambda s: (s % NB, 0))


def _full_spec(shape):
    return pl.BlockSpec(shape, lambda s: (0,) * len(shape))


@jax.jit
def kernel(x, A_pos, A_neg, W_in, b_in, W_pos0, b_pos0, W_neg0, b_neg0,
           W_pos1, b_pos1, W_neg1, b_neg1):
    f32 = jnp.float32
    return pl.pallas_call(
        _gcn_kernel,
        grid=(2 * NB,),
        in_specs=[
            _full_spec((N, H)),
            _stripe_spec(),
            _stripe_spec(),
            _full_spec((H, H)),
            _full_spec((1, H)),
            _full_spec((H, H)),
            _full_spec((1, H)),
            _full_spec((H, H)),
            _full_spec((1, H)),
            _full_spec((H, H)),
            _full_spec((1, H)),
            _full_spec((H, H)),
            _full_spec((1, H)),
        ],
        out_specs=pl.BlockSpec((BM, H),
                               lambda s: (jnp.maximum(s - NB, 0), 0)),
        out_shape=jax.ShapeDtypeStruct((N, H), f32),
        scratch_shapes=[
            pltpu.VMEM((N, H), f32),
            pltpu.VMEM((N, H), f32),
        ],
    )(x, A_pos, A_neg, W_in, b_in.reshape(1, H),
      W_pos0, b_pos0.reshape(1, H), W_neg0, b_neg0.reshape(1, H),
      W_pos1, b_pos1.reshape(1, H), W_neg1, b_neg1.reshape(1, H))
